# baseline (speedup 1.0000x reference)
import functools

import jax
import jax.numpy as jnp
from jax import lax
from jax.experimental import pallas as pl
from jax.experimental.pallas import tpu as pltpu
from jax.experimental.pallas import tpu_sc as plsc

N_NODES = 50000
N_BATCH = 1024
ROWS = N_NODES + N_BATCH
ROWS_PAD = 51072
D = 64
H = 32
E_EDGES = 800000
E_TOTAL = E_EDGES + N_NODES
NUM_TILES = 16
CHUNK = 416
EP = 851968
TE = EP // NUM_TILES
ITERS = TE // CHUNK
RPT = ROWS_PAD // NUM_TILES

BLK = 3192
GRID = ROWS_PAD // BLK


def _make_agg():
    mesh = plsc.VectorSubcoreMesh(core_axis_name="c", subcore_axis_name="s")

    @functools.partial(
        pl.kernel,
        mesh=mesh,
        compiler_params=pltpu.CompilerParams(use_tc_tiling_on_sc=False),
        out_type=[
            jax.ShapeDtypeStruct((ROWS_PAD, H), jnp.float32),
            jax.ShapeDtypeStruct((ROWS_PAD, H), jnp.float32),
        ],
        scratch_types=[
            pltpu.VMEM((CHUNK,), jnp.int32),
            pltpu.VMEM((CHUNK,), jnp.int32),
            pltpu.VMEM((CHUNK,), jnp.int32),
            pltpu.VMEM((CHUNK,), jnp.int32),
            pltpu.VMEM((CHUNK, H), jnp.float32),
            pltpu.VMEM((CHUNK, H), jnp.float32),
            pltpu.VMEM_SHARED((ROWS_PAD, H), jnp.float32),
            pltpu.SemaphoreType.DMA,
            pltpu.SemaphoreType.DMA,
            pltpu.SemaphoreType.DMA,
            pltpu.SemaphoreType.DMA,
            pltpu.SemaphoreType.DMA,
            pltpu.SemaphoreType.DMA,
        ],
    )
    def agg(h0_hbm, h1_hbm, src_hbm, dst_hbm, zeros_hbm,
            out0, out1,
            sv0, sv1, dv0, dv1, rw0, rw1, acc,
            si0, si1, sg0, sg1, ss0, ss1):
        c = lax.axis_index("c")
        s = lax.axis_index("s")
        srcv = (sv0, sv1)
        dstv = (dv0, dv1)
        rows = (rw0, rw1)
        s_idx = (si0, si1)
        s_g = (sg0, sg1)
        s_s = (ss0, ss1)
        base = s * TE

        def load_idx(i, k):
            e0 = base + i * CHUNK
            pltpu.async_copy(src_hbm.at[pl.ds(e0, CHUNK)], srcv[k], s_idx[k])
            pltpu.async_copy(dst_hbm.at[pl.ds(e0, CHUNK)], dstv[k], s_idx[k])

        load_idx(0, 0)
        pltpu.sync_copy(zeros_hbm.at[pl.ds(s * RPT, RPT)],
                        acc.at[pl.ds(s * RPT, RPT)])
        plsc.subcore_barrier()

        @pl.loop(0, ITERS, step=2)
        def _(i0):
            for j in range(2):
                i = i0 + j
                k = j % 2
                o = (j + 1) % 2

                e0 = base + i * CHUNK
                pltpu.make_async_copy(
                    src_hbm.at[pl.ds(e0, CHUNK)], srcv[k], s_idx[k]).wait()
                pltpu.make_async_copy(
                    dst_hbm.at[pl.ds(e0, CHUNK)], dstv[k], s_idx[k]).wait()

                @pl.when(c == 0)
                def _():
                    pltpu.async_copy(h0_hbm.at[srcv[k]], rows[k], s_g[k])

                @pl.when(c == 1)
                def _():
                    pltpu.async_copy(h1_hbm.at[srcv[k]], rows[k], s_g[k])

                @pl.when(i >= 1)
                def _():
                    pltpu.make_async_copy(
                        rows[o], acc.at[dstv[o]], s_s[o]).wait()

                @pl.when(i + 1 < ITERS)
                def _():
                    load_idx(i + 1, o)

                @pl.when(c == 0)
                def _():
                    pltpu.make_async_copy(
                        h0_hbm.at[srcv[k]], rows[k], s_g[k]).wait()

                @pl.when(c == 1)
                def _():
                    pltpu.make_async_copy(
                        h1_hbm.at[srcv[k]], rows[k], s_g[k]).wait()

                pltpu.async_copy(rows[k], acc.at[dstv[k]], s_s[k], add=True)

        pltpu.make_async_copy(rows[1], acc.at[dstv[1]], s_s[1]).wait()
        plsc.subcore_barrier()

        @pl.when(c == 0)
        def _():
            pltpu.sync_copy(acc.at[pl.ds(s * RPT, RPT)],
                            out0.at[pl.ds(s * RPT, RPT)])

        @pl.when(c == 1)
        def _():
            pltpu.sync_copy(acc.at[pl.ds(s * RPT, RPT)],
                            out1.at[pl.ds(s * RPT, RPT)])

    return agg


_agg = _make_agg()


def _normalize_rows(z):
    n = jnp.sqrt(jnp.sum(z * z, axis=1, keepdims=True))
    return z / jnp.maximum(n, 1e-12)


def _init_body(x_ref, w1_ref, olo_ref, ohi_ref):
    z = jnp.dot(x_ref[...], w1_ref[...], preferred_element_type=jnp.float32)
    z = _normalize_rows(jnp.maximum(z, 0.0))
    olo_ref[...] = z[:, :H]
    ohi_ref[...] = z[:, H:]


def _init_call(x, w1):
    return pl.pallas_call(
        _init_body,
        grid=(GRID,),
        in_specs=[
            pl.BlockSpec((BLK, 2), lambda i: (i, 0)),
            pl.BlockSpec((2, D), lambda i: (0, 0)),
        ],
        out_specs=[
            pl.BlockSpec((BLK, H), lambda i: (i, 0)),
            pl.BlockSpec((BLK, H), lambda i: (i, 0)),
        ],
        out_shape=[
            jax.ShapeDtypeStruct((ROWS_PAD, H), jnp.float32),
            jax.ShapeDtypeStruct((ROWS_PAD, H), jnp.float32),
        ],
    )(x, w1)


def _dense_body(hlo_ref, hhi_ref, alo_ref, ahi_ref, w2_ref, w3_ref,
                lw_ref, b_ref, olo_ref, ohi_ref):
    h = jnp.concatenate([hlo_ref[...], hhi_ref[...]], axis=1)
    a = jnp.concatenate([alo_ref[...], ahi_ref[...]], axis=1)
    m2 = jnp.dot(w2_ref[...], lw_ref[:D, :], preferred_element_type=jnp.float32)
    m3 = jnp.dot(w3_ref[...], lw_ref[D:, :], preferred_element_type=jnp.float32)
    z = (jnp.dot(h, m2, preferred_element_type=jnp.float32)
         + jnp.dot(a, m3, preferred_element_type=jnp.float32)
         + b_ref[...])
    z = _normalize_rows(jnp.maximum(z, 0.0))
    olo_ref[...] = z[:, :H]
    ohi_ref[...] = z[:, H:]


def _dense_call(h_lo, h_hi, a_lo, a_hi, w2, w3, lw, b2d):
    return pl.pallas_call(
        _dense_body,
        grid=(GRID,),
        in_specs=[
            pl.BlockSpec((BLK, H), lambda i: (i, 0)),
            pl.BlockSpec((BLK, H), lambda i: (i, 0)),
            pl.BlockSpec((BLK, H), lambda i: (i, 0)),
            pl.BlockSpec((BLK, H), lambda i: (i, 0)),
            pl.BlockSpec((D, D), lambda i: (0, 0)),
            pl.BlockSpec((D, D), lambda i: (0, 0)),
            pl.BlockSpec((2 * D, D), lambda i: (0, 0)),
            pl.BlockSpec((1, D), lambda i: (0, 0)),
        ],
        out_specs=[
            pl.BlockSpec((BLK, H), lambda i: (i, 0)),
            pl.BlockSpec((BLK, H), lambda i: (i, 0)),
        ],
        out_shape=[
            jax.ShapeDtypeStruct((ROWS_PAD, H), jnp.float32),
            jax.ShapeDtypeStruct((ROWS_PAD, H), jnp.float32),
        ],
    )(h_lo, h_hi, a_lo, a_hi, w2, w3, lw, b2d)


def kernel(input_features, input_feature_s, W1, W2, W3, lin_W, lin_b,
           edge_index, batch_ids, depth):
    dst = edge_index[0].astype(jnp.int32)
    src = edge_index[1].astype(jnp.int32)

    seg_src = jnp.arange(N_NODES, dtype=jnp.int32)
    seg_dst = batch_ids.astype(jnp.int32) + N_NODES
    pad = EP - E_TOTAL
    src_full = jnp.concatenate([src, seg_src,
                                jnp.zeros((pad,), jnp.int32)])
    dst_full = jnp.concatenate([dst, seg_dst,
                                jnp.full((pad,), ROWS, jnp.int32)])
    zeros = jnp.zeros((ROWS_PAD, H), jnp.float32)

    x_full = jnp.concatenate([
        input_features.astype(jnp.float32),
        input_feature_s.astype(jnp.float32),
        jnp.zeros((ROWS_PAD - ROWS, 2), jnp.float32),
    ])
    b2d = lin_b.reshape(1, D).astype(jnp.float32)

    h_lo, h_hi = _init_call(x_full, W1.astype(jnp.float32))

    def step(_, carry):
        h_lo, h_hi = carry
        a_lo, a_hi = _agg(h_lo, h_hi, src_full, dst_full, zeros)
        return tuple(_dense_call(h_lo, h_hi, a_lo, a_hi, W2, W3, lin_W, b2d))

    h_lo, h_hi = lax.fori_loop(0, depth, step, (h_lo, h_hi))
    hh = jnp.concatenate([h_lo, h_hi], axis=1)
    return hh[:N_NODES], hh[N_NODES:ROWS]


# speedup vs baseline: 1.3923x; 1.3923x over previous
import functools

import jax
import jax.numpy as jnp
from jax import lax
from jax.experimental import pallas as pl
from jax.experimental.pallas import tpu as pltpu
from jax.experimental.pallas import tpu_sc as plsc

N_NODES = 50000
N_BATCH = 1024
ROWS = N_NODES + N_BATCH
ROWS_PAD = 51072
D = 64
H = 32
E_EDGES = 800000
E_TOTAL = E_EDGES + N_NODES
NUM_TILES = 16
CHUNK = 416
EP = 851968
TE = EP // NUM_TILES
ITERS = TE // CHUNK
RPT = ROWS_PAD // NUM_TILES

RP4 = ROWS_PAD // 4
BLK4 = 912
GRID = RP4 // BLK4


def _make_agg():
    mesh = plsc.VectorSubcoreMesh(core_axis_name="c", subcore_axis_name="s")

    @functools.partial(
        pl.kernel,
        mesh=mesh,
        compiler_params=pltpu.CompilerParams(use_tc_tiling_on_sc=False),
        out_type=[
            jax.ShapeDtypeStruct((ROWS_PAD, H), jnp.float32),
            jax.ShapeDtypeStruct((ROWS_PAD, H), jnp.float32),
        ],
        scratch_types=[
            pltpu.VMEM((CHUNK,), jnp.int32),
            pltpu.VMEM((CHUNK,), jnp.int32),
            pltpu.VMEM((CHUNK,), jnp.int32),
            pltpu.VMEM((CHUNK,), jnp.int32),
            pltpu.VMEM((CHUNK, H), jnp.float32),
            pltpu.VMEM((CHUNK, H), jnp.float32),
            pltpu.VMEM_SHARED((ROWS_PAD, H), jnp.float32),
            pltpu.SemaphoreType.DMA,
            pltpu.SemaphoreType.DMA,
            pltpu.SemaphoreType.DMA,
            pltpu.SemaphoreType.DMA,
            pltpu.SemaphoreType.DMA,
            pltpu.SemaphoreType.DMA,
        ],
    )
    def agg(h0_hbm, h1_hbm, src_hbm, dst_hbm, zeros_hbm,
            out0, out1,
            sv0, sv1, dv0, dv1, rw0, rw1, acc,
            si0, si1, sg0, sg1, ss0, ss1):
        c = lax.axis_index("c")
        s = lax.axis_index("s")
        srcv = (sv0, sv1)
        dstv = (dv0, dv1)
        rows = (rw0, rw1)
        s_idx = (si0, si1)
        s_g = (sg0, sg1)
        s_s = (ss0, ss1)
        base = s * TE

        def load_idx(i, k):
            e0 = base + i * CHUNK
            pltpu.async_copy(src_hbm.at[pl.ds(e0, CHUNK)], srcv[k], s_idx[k])
            pltpu.async_copy(dst_hbm.at[pl.ds(e0, CHUNK)], dstv[k], s_idx[k])

        load_idx(0, 0)
        pltpu.sync_copy(zeros_hbm.at[pl.ds(s * RPT, RPT)],
                        acc.at[pl.ds(s * RPT, RPT)])
        plsc.subcore_barrier()

        @pl.loop(0, ITERS, step=2)
        def _(i0):
            for j in range(2):
                i = i0 + j
                k = j % 2
                o = (j + 1) % 2

                e0 = base + i * CHUNK
                pltpu.make_async_copy(
                    src_hbm.at[pl.ds(e0, CHUNK)], srcv[k], s_idx[k]).wait()
                pltpu.make_async_copy(
                    dst_hbm.at[pl.ds(e0, CHUNK)], dstv[k], s_idx[k]).wait()

                @pl.when(c == 0)
                def _():
                    pltpu.async_copy(h0_hbm.at[srcv[k]], rows[k], s_g[k])

                @pl.when(c == 1)
                def _():
                    pltpu.async_copy(h1_hbm.at[srcv[k]], rows[k], s_g[k])

                @pl.when(i >= 1)
                def _():
                    pltpu.make_async_copy(
                        rows[o], acc.at[dstv[o]], s_s[o]).wait()

                @pl.when(i + 1 < ITERS)
                def _():
                    load_idx(i + 1, o)

                @pl.when(c == 0)
                def _():
                    pltpu.make_async_copy(
                        h0_hbm.at[srcv[k]], rows[k], s_g[k]).wait()

                @pl.when(c == 1)
                def _():
                    pltpu.make_async_copy(
                        h1_hbm.at[srcv[k]], rows[k], s_g[k]).wait()

                pltpu.async_copy(rows[k], acc.at[dstv[k]], s_s[k], add=True)

        pltpu.make_async_copy(rows[1], acc.at[dstv[1]], s_s[1]).wait()
        plsc.subcore_barrier()

        @pl.when(c == 0)
        def _():
            pltpu.sync_copy(acc.at[pl.ds(s * RPT, RPT)],
                            out0.at[pl.ds(s * RPT, RPT)])

        @pl.when(c == 1)
        def _():
            pltpu.sync_copy(acc.at[pl.ds(s * RPT, RPT)],
                            out1.at[pl.ds(s * RPT, RPT)])

    return agg


_agg = _make_agg()


def _init_body(x_ref, wlo_ref, whi_ref, ones_ref, blo_ref, bhi_ref,
               olo_ref, ohi_ref):
    x = x_ref[...]
    zl = jnp.dot(x, wlo_ref[...], preferred_element_type=jnp.float32)
    zh = jnp.dot(x, whi_ref[...], preferred_element_type=jnp.float32)
    zl = jnp.maximum(zl + blo_ref[...], 0.0)
    zh = jnp.maximum(zh + bhi_ref[...], 0.0)
    s = jnp.dot(zl * zl + zh * zh, ones_ref[...],
                preferred_element_type=jnp.float32)
    inv = 1.0 / jnp.maximum(jnp.sqrt(s), 1e-12)
    olo_ref[...] = zl * inv
    ohi_ref[...] = zh * inv


def _init_call(xp, wlo, whi, ones_bd, blo, bhi):
    return pl.pallas_call(
        _init_body,
        grid=(GRID,),
        in_specs=[
            pl.BlockSpec((BLK4, 8), lambda i: (i, 0)),
            pl.BlockSpec((8, 128), lambda i: (0, 0)),
            pl.BlockSpec((8, 128), lambda i: (0, 0)),
            pl.BlockSpec((128, 128), lambda i: (0, 0)),
            pl.BlockSpec((1, 128), lambda i: (0, 0)),
            pl.BlockSpec((1, 128), lambda i: (0, 0)),
        ],
        out_specs=[
            pl.BlockSpec((BLK4, 128), lambda i: (i, 0)),
            pl.BlockSpec((BLK4, 128), lambda i: (i, 0)),
        ],
        out_shape=[
            jax.ShapeDtypeStruct((RP4, 128), jnp.float32),
            jax.ShapeDtypeStruct((RP4, 128), jnp.float32),
        ],
    )(xp, wlo, whi, ones_bd, blo, bhi)


def _dense_body(hlo_ref, hhi_ref, alo_ref, ahi_ref, wlo_ref, whi_ref,
                ones_ref, blo_ref, bhi_ref, olo_ref, ohi_ref):
    x4 = jnp.concatenate(
        [hlo_ref[...], hhi_ref[...], alo_ref[...], ahi_ref[...]], axis=1)
    zl = jnp.dot(x4, wlo_ref[...], preferred_element_type=jnp.float32)
    zh = jnp.dot(x4, whi_ref[...], preferred_element_type=jnp.float32)
    zl = jnp.maximum(zl + blo_ref[...], 0.0)
    zh = jnp.maximum(zh + bhi_ref[...], 0.0)
    s = jnp.dot(zl * zl + zh * zh, ones_ref[...],
                preferred_element_type=jnp.float32)
    inv = 1.0 / jnp.maximum(jnp.sqrt(s), 1e-12)
    olo_ref[...] = zl * inv
    ohi_ref[...] = zh * inv


def _dense_call(h_lo, h_hi, a_lo, a_hi, wlo, whi, ones_bd, blo, bhi):
    return pl.pallas_call(
        _dense_body,
        grid=(GRID,),
        in_specs=[
            pl.BlockSpec((BLK4, 128), lambda i: (i, 0)),
            pl.BlockSpec((BLK4, 128), lambda i: (i, 0)),
            pl.BlockSpec((BLK4, 128), lambda i: (i, 0)),
            pl.BlockSpec((BLK4, 128), lambda i: (i, 0)),
            pl.BlockSpec((512, 128), lambda i: (0, 0)),
            pl.BlockSpec((512, 128), lambda i: (0, 0)),
            pl.BlockSpec((128, 128), lambda i: (0, 0)),
            pl.BlockSpec((1, 128), lambda i: (0, 0)),
            pl.BlockSpec((1, 128), lambda i: (0, 0)),
        ],
        out_specs=[
            pl.BlockSpec((BLK4, 128), lambda i: (i, 0)),
            pl.BlockSpec((BLK4, 128), lambda i: (i, 0)),
        ],
        out_shape=[
            jax.ShapeDtypeStruct((RP4, 128), jnp.float32),
            jax.ShapeDtypeStruct((RP4, 128), jnp.float32),
        ],
    )(h_lo, h_hi, a_lo, a_hi, wlo, whi, ones_bd, blo, bhi)


def kernel(input_features, input_feature_s, W1, W2, W3, lin_W, lin_b,
           edge_index, batch_ids, depth):
    f32 = jnp.float32
    dst = edge_index[0].astype(jnp.int32)
    src = edge_index[1].astype(jnp.int32)

    seg_src = jnp.arange(N_NODES, dtype=jnp.int32)
    seg_dst = batch_ids.astype(jnp.int32) + N_NODES
    pad = EP - E_TOTAL
    src_full = jnp.concatenate([src, seg_src,
                                jnp.zeros((pad,), jnp.int32)])
    dst_full = jnp.concatenate([dst, seg_dst,
                                jnp.full((pad,), ROWS, jnp.int32)])
    zeros = jnp.zeros((ROWS_PAD, H), f32)

    x_full = jnp.concatenate([
        input_features.astype(f32),
        input_feature_s.astype(f32),
        jnp.zeros((ROWS_PAD - ROWS, 2), f32),
    ])
    xp = x_full.reshape(RP4, 8)

    eye4 = jnp.eye(4, dtype=f32)

    def bd(k):
        return jnp.kron(eye4, k.astype(f32))

    W1f = W1.astype(f32)
    w1lo = bd(W1f[:, :H])
    w1hi = bd(W1f[:, H:])
    m2 = jnp.dot(W2.astype(f32), lin_W.astype(f32)[:D])
    m3 = jnp.dot(W3.astype(f32), lin_W.astype(f32)[D:])
    wlo = jnp.concatenate(
        [bd(m2[:H, :H]), bd(m2[H:, :H]), bd(m3[:H, :H]), bd(m3[H:, :H])],
        axis=0)
    whi = jnp.concatenate(
        [bd(m2[:H, H:]), bd(m2[H:, H:]), bd(m3[:H, H:]), bd(m3[H:, H:])],
        axis=0)
    ones_bd = bd(jnp.ones((H, H), f32))
    blo = jnp.tile(lin_b.astype(f32)[:H], 4).reshape(1, 128)
    bhi = jnp.tile(lin_b.astype(f32)[H:], 4).reshape(1, 128)
    zb = jnp.zeros((1, 128), f32)

    h_lo, h_hi = _init_call(xp, w1lo, w1hi, ones_bd, zb, zb)

    def step(_, carry):
        h_lo, h_hi = carry
        a_lo, a_hi = _agg(h_lo.reshape(ROWS_PAD, H), h_hi.reshape(ROWS_PAD, H),
                          src_full, dst_full, zeros)
        return tuple(_dense_call(h_lo, h_hi,
                                 a_lo.reshape(RP4, 128),
                                 a_hi.reshape(RP4, 128),
                                 wlo, whi, ones_bd, blo, bhi))

    h_lo, h_hi = lax.fori_loop(0, depth, step, (h_lo, h_hi))
    hh = jnp.concatenate([h_lo.reshape(ROWS_PAD, H),
                          h_hi.reshape(ROWS_PAD, H)], axis=1)
    return hh[:N_NODES], hh[N_NODES:ROWS]


# speedup vs baseline: 1.4510x; 1.0421x over previous
import functools

import jax
import jax.numpy as jnp
from jax import lax
from jax.experimental import pallas as pl
from jax.experimental.pallas import tpu as pltpu
from jax.experimental.pallas import tpu_sc as plsc

N_NODES = 50000
N_BATCH = 1024
ROWS = N_NODES + N_BATCH
ROWS_PAD = 51072
D = 64
H = 32
E_EDGES = 800000
E_TOTAL = E_EDGES + N_NODES
NUM_TILES = 16
CHUNK = 416
EP = 851968
TE = EP // NUM_TILES
ITERS = TE // CHUNK
RPT = ROWS_PAD // NUM_TILES

RP4 = ROWS_PAD // 4
BLK4 = 912
GRID = RP4 // BLK4


def _make_agg():
    mesh = plsc.VectorSubcoreMesh(core_axis_name="c", subcore_axis_name="s")

    @functools.partial(
        pl.kernel,
        mesh=mesh,
        compiler_params=pltpu.CompilerParams(use_tc_tiling_on_sc=False),
        out_type=[
            jax.ShapeDtypeStruct((ROWS_PAD, H), jnp.float32),
            jax.ShapeDtypeStruct((ROWS_PAD, H), jnp.float32),
        ],
        scratch_types=[
            pltpu.VMEM((CHUNK,), jnp.int32),
            pltpu.VMEM((CHUNK,), jnp.int32),
            pltpu.VMEM((CHUNK,), jnp.int32),
            pltpu.VMEM((CHUNK,), jnp.int32),
            pltpu.VMEM((CHUNK, H), jnp.float32),
            pltpu.VMEM((CHUNK, H), jnp.float32),
            pltpu.VMEM_SHARED((ROWS_PAD, H), jnp.float32),
            pltpu.SemaphoreType.DMA,
            pltpu.SemaphoreType.DMA,
            pltpu.SemaphoreType.DMA,
            pltpu.SemaphoreType.DMA,
            pltpu.SemaphoreType.DMA,
            pltpu.SemaphoreType.DMA,
        ],
    )
    def agg(h0_hbm, h1_hbm, src_hbm, dst_hbm, zeros_hbm,
            out0, out1,
            sv0, sv1, dv0, dv1, rw0, rw1, acc,
            si0, si1, sg0, sg1, ss0, ss1):
        c = lax.axis_index("c")
        s = lax.axis_index("s")
        srcv = (sv0, sv1)
        dstv = (dv0, dv1)
        rows = (rw0, rw1)
        s_idx = (si0, si1)
        s_g = (sg0, sg1)
        s_s = (ss0, ss1)
        base = s * TE

        def load_idx(i, k):
            e0 = base + i * CHUNK
            pltpu.async_copy(src_hbm.at[pl.ds(e0, CHUNK)], srcv[k], s_idx[k])
            pltpu.async_copy(dst_hbm.at[pl.ds(e0, CHUNK)], dstv[k], s_idx[k])

        load_idx(0, 0)
        pltpu.sync_copy(zeros_hbm.at[pl.ds(s * RPT, RPT)],
                        acc.at[pl.ds(s * RPT, RPT)])
        plsc.subcore_barrier()

        @pl.loop(0, ITERS, step=2)
        def _(i0):
            for j in range(2):
                i = i0 + j
                k = j % 2
                o = (j + 1) % 2

                e0 = base + i * CHUNK
                pltpu.make_async_copy(
                    src_hbm.at[pl.ds(e0, CHUNK)], srcv[k], s_idx[k]).wait()
                pltpu.make_async_copy(
                    dst_hbm.at[pl.ds(e0, CHUNK)], dstv[k], s_idx[k]).wait()

                @pl.when(c == 0)
                def _():
                    pltpu.async_copy(h0_hbm.at[srcv[k]], rows[k], s_g[k])

                @pl.when(c == 1)
                def _():
                    pltpu.async_copy(h1_hbm.at[srcv[k]], rows[k], s_g[k])

                @pl.when(i >= 1)
                def _():
                    pltpu.make_async_copy(
                        rows[o], acc.at[dstv[o]], s_s[o]).wait()

                @pl.when(i + 1 < ITERS)
                def _():
                    load_idx(i + 1, o)

                @pl.when(c == 0)
                def _():
                    pltpu.make_async_copy(
                        h0_hbm.at[srcv[k]], rows[k], s_g[k]).wait()

                @pl.when(c == 1)
                def _():
                    pltpu.make_async_copy(
                        h1_hbm.at[srcv[k]], rows[k], s_g[k]).wait()

                pltpu.async_copy(rows[k], acc.at[dstv[k]], s_s[k], add=True)

        pltpu.make_async_copy(rows[1], acc.at[dstv[1]], s_s[1]).wait()
        plsc.subcore_barrier()

        @pl.when(c == 0)
        def _():
            pltpu.sync_copy(acc.at[pl.ds(s * RPT, RPT)],
                            out0.at[pl.ds(s * RPT, RPT)])

        @pl.when(c == 1)
        def _():
            pltpu.sync_copy(acc.at[pl.ds(s * RPT, RPT)],
                            out1.at[pl.ds(s * RPT, RPT)])

    return agg


_agg = _make_agg()


def _init_body(x_ref, wlo_ref, whi_ref, ones_ref, blo_ref, bhi_ref,
               olo_ref, ohi_ref):
    x = x_ref[...]
    zl = jnp.dot(x, wlo_ref[...], preferred_element_type=jnp.float32)
    zh = jnp.dot(x, whi_ref[...], preferred_element_type=jnp.float32)
    zl = jnp.maximum(zl + blo_ref[...], 0.0)
    zh = jnp.maximum(zh + bhi_ref[...], 0.0)
    s = jnp.dot(zl * zl + zh * zh, ones_ref[...],
                preferred_element_type=jnp.float32)
    inv = 1.0 / jnp.maximum(jnp.sqrt(s), 1e-12)
    olo_ref[...] = zl * inv
    ohi_ref[...] = zh * inv


def _init_call(xp, wlo, whi, ones_bd, blo, bhi):
    return pl.pallas_call(
        _init_body,
        grid=(GRID,),
        in_specs=[
            pl.BlockSpec((BLK4, 8), lambda i: (i, 0)),
            pl.BlockSpec((8, 128), lambda i: (0, 0)),
            pl.BlockSpec((8, 128), lambda i: (0, 0)),
            pl.BlockSpec((128, 128), lambda i: (0, 0)),
            pl.BlockSpec((1, 128), lambda i: (0, 0)),
            pl.BlockSpec((1, 128), lambda i: (0, 0)),
        ],
        out_specs=[
            pl.BlockSpec((BLK4, 128), lambda i: (i, 0)),
            pl.BlockSpec((BLK4, 128), lambda i: (i, 0)),
        ],
        out_shape=[
            jax.ShapeDtypeStruct((RP4, 128), jnp.float32),
            jax.ShapeDtypeStruct((RP4, 128), jnp.float32),
        ],
    )(xp, wlo, whi, ones_bd, blo, bhi)


def _dense_body(hlo_ref, hhi_ref, alo_ref, ahi_ref, wlo_ref, whi_ref,
                ones_ref, blo_ref, bhi_ref, olo_ref, ohi_ref):
    x4 = jnp.concatenate(
        [hlo_ref[...], hhi_ref[...], alo_ref[...], ahi_ref[...]], axis=1)
    zl = jnp.dot(x4, wlo_ref[...], preferred_element_type=jnp.float32)
    zh = jnp.dot(x4, whi_ref[...], preferred_element_type=jnp.float32)
    zl = jnp.maximum(zl + blo_ref[...], 0.0)
    zh = jnp.maximum(zh + bhi_ref[...], 0.0)
    s = jnp.dot(zl * zl + zh * zh, ones_ref[...],
                preferred_element_type=jnp.float32)
    inv = 1.0 / jnp.maximum(jnp.sqrt(s), 1e-12)
    olo_ref[...] = zl * inv
    ohi_ref[...] = zh * inv


def _dense_call(h_lo, h_hi, a_lo, a_hi, wlo, whi, ones_bd, blo, bhi):
    return pl.pallas_call(
        _dense_body,
        grid=(GRID,),
        in_specs=[
            pl.BlockSpec((BLK4, 128), lambda i: (i, 0)),
            pl.BlockSpec((BLK4, 128), lambda i: (i, 0)),
            pl.BlockSpec((BLK4, 128), lambda i: (i, 0)),
            pl.BlockSpec((BLK4, 128), lambda i: (i, 0)),
            pl.BlockSpec((512, 128), lambda i: (0, 0)),
            pl.BlockSpec((512, 128), lambda i: (0, 0)),
            pl.BlockSpec((128, 128), lambda i: (0, 0)),
            pl.BlockSpec((1, 128), lambda i: (0, 0)),
            pl.BlockSpec((1, 128), lambda i: (0, 0)),
        ],
        out_specs=[
            pl.BlockSpec((BLK4, 128), lambda i: (i, 0)),
            pl.BlockSpec((BLK4, 128), lambda i: (i, 0)),
        ],
        out_shape=[
            jax.ShapeDtypeStruct((RP4, 128), jnp.float32),
            jax.ShapeDtypeStruct((RP4, 128), jnp.float32),
        ],
    )(h_lo, h_hi, a_lo, a_hi, wlo, whi, ones_bd, blo, bhi)


def kernel(input_features, input_feature_s, W1, W2, W3, lin_W, lin_b,
           edge_index, batch_ids, depth):
    f32 = jnp.float32
    dst = edge_index[0].astype(jnp.int32)
    src = edge_index[1].astype(jnp.int32)

    pad = EP - E_TOTAL
    seg_src = jnp.minimum(jnp.arange(N_NODES + pad, dtype=jnp.int32),
                          N_NODES - 1)
    seg_dst = jnp.concatenate([batch_ids.astype(jnp.int32) + N_NODES,
                               jnp.full((pad,), ROWS, jnp.int32)])
    src_full = jnp.concatenate(
        [src.reshape(E_EDGES // 128, 128),
         seg_src.reshape((N_NODES + pad) // 128, 128)]).reshape(EP)
    dst_full = jnp.concatenate(
        [dst.reshape(E_EDGES // 128, 128),
         seg_dst.reshape((N_NODES + pad) // 128, 128)]).reshape(EP)
    zeros = jnp.zeros((ROWS_PAD, H), f32)

    x_full = jnp.concatenate([
        input_features.astype(f32),
        input_feature_s.astype(f32),
        jnp.zeros((ROWS_PAD - ROWS, 2), f32),
    ])
    xp = x_full.reshape(RP4, 8)

    eye4 = jnp.eye(4, dtype=f32)

    def bd(k):
        return jnp.kron(eye4, k.astype(f32))

    W1f = W1.astype(f32)
    w1lo = bd(W1f[:, :H])
    w1hi = bd(W1f[:, H:])
    m2 = jnp.dot(W2.astype(f32), lin_W.astype(f32)[:D])
    m3 = jnp.dot(W3.astype(f32), lin_W.astype(f32)[D:])
    wlo = jnp.concatenate(
        [bd(m2[:H, :H]), bd(m2[H:, :H]), bd(m3[:H, :H]), bd(m3[H:, :H])],
        axis=0)
    whi = jnp.concatenate(
        [bd(m2[:H, H:]), bd(m2[H:, H:]), bd(m3[:H, H:]), bd(m3[H:, H:])],
        axis=0)
    ones_bd = bd(jnp.ones((H, H), f32))
    blo = jnp.tile(lin_b.astype(f32)[:H], 4).reshape(1, 128)
    bhi = jnp.tile(lin_b.astype(f32)[H:], 4).reshape(1, 128)
    zb = jnp.zeros((1, 128), f32)

    h_lo, h_hi = _init_call(xp, w1lo, w1hi, ones_bd, zb, zb)

    def step(_, carry):
        h_lo, h_hi = carry
        a_lo, a_hi = _agg(h_lo.reshape(ROWS_PAD, H), h_hi.reshape(ROWS_PAD, H),
                          src_full, dst_full, zeros)
        return tuple(_dense_call(h_lo, h_hi,
                                 a_lo.reshape(RP4, 128),
                                 a_hi.reshape(RP4, 128),
                                 wlo, whi, ones_bd, blo, bhi))

    h_lo, h_hi = lax.fori_loop(0, depth, step, (h_lo, h_hi))
    hh = jnp.concatenate([h_lo.reshape(ROWS_PAD, H),
                          h_hi.reshape(ROWS_PAD, H)], axis=1)
    return hh[:N_NODES], hh[N_NODES:ROWS]


# speedup vs baseline: 1.4795x; 1.0197x over previous
import functools

import jax
import jax.numpy as jnp
from jax import lax
from jax.experimental import pallas as pl
from jax.experimental.pallas import tpu as pltpu
from jax.experimental.pallas import tpu_sc as plsc

N_NODES = 50000
N_BATCH = 1024
ROWS = N_NODES + N_BATCH
ROWS_PAD = 51072
D = 64
H = 32
E_EDGES = 800000
E_TOTAL = E_EDGES + N_NODES
NUM_TILES = 16
CHUNK = 416
EP = 851968
TE = EP // NUM_TILES
ITERS = TE // CHUNK
RPT = ROWS_PAD // NUM_TILES

RP4 = ROWS_PAD // 4
BLK4 = 1824
GRID = RP4 // BLK4


def _make_agg():
    mesh = plsc.VectorSubcoreMesh(core_axis_name="c", subcore_axis_name="s")

    @functools.partial(
        pl.kernel,
        mesh=mesh,
        compiler_params=pltpu.CompilerParams(use_tc_tiling_on_sc=False),
        out_type=[
            jax.ShapeDtypeStruct((ROWS_PAD, H), jnp.float32),
            jax.ShapeDtypeStruct((ROWS_PAD, H), jnp.float32),
        ],
        scratch_types=[
            pltpu.VMEM((CHUNK,), jnp.int32),
            pltpu.VMEM((CHUNK,), jnp.int32),
            pltpu.VMEM((CHUNK,), jnp.int32),
            pltpu.VMEM((CHUNK,), jnp.int32),
            pltpu.VMEM((CHUNK, H), jnp.float32),
            pltpu.VMEM((CHUNK, H), jnp.float32),
            pltpu.VMEM_SHARED((ROWS_PAD, H), jnp.float32),
            pltpu.SemaphoreType.DMA,
            pltpu.SemaphoreType.DMA,
            pltpu.SemaphoreType.DMA,
            pltpu.SemaphoreType.DMA,
            pltpu.SemaphoreType.DMA,
            pltpu.SemaphoreType.DMA,
        ],
    )
    def agg(h0_hbm, h1_hbm, src_hbm, dst_hbm, zeros_hbm,
            out0, out1,
            sv0, sv1, dv0, dv1, rw0, rw1, acc,
            si0, si1, sg0, sg1, ss0, ss1):
        c = lax.axis_index("c")
        s = lax.axis_index("s")
        srcv = (sv0, sv1)
        dstv = (dv0, dv1)
        rows = (rw0, rw1)
        s_idx = (si0, si1)
        s_g = (sg0, sg1)
        s_s = (ss0, ss1)
        base = s * TE

        def load_idx(i, k):
            e0 = base + i * CHUNK
            pltpu.async_copy(src_hbm.at[pl.ds(e0, CHUNK)], srcv[k], s_idx[k])
            pltpu.async_copy(dst_hbm.at[pl.ds(e0, CHUNK)], dstv[k], s_idx[k])

        load_idx(0, 0)
        pltpu.sync_copy(zeros_hbm.at[pl.ds(s * RPT, RPT)],
                        acc.at[pl.ds(s * RPT, RPT)])
        plsc.subcore_barrier()

        @pl.loop(0, ITERS, step=2)
        def _(i0):
            for j in range(2):
                i = i0 + j
                k = j % 2
                o = (j + 1) % 2

                e0 = base + i * CHUNK
                pltpu.make_async_copy(
                    src_hbm.at[pl.ds(e0, CHUNK)], srcv[k], s_idx[k]).wait()
                pltpu.make_async_copy(
                    dst_hbm.at[pl.ds(e0, CHUNK)], dstv[k], s_idx[k]).wait()

                @pl.when(c == 0)
                def _():
                    pltpu.async_copy(h0_hbm.at[srcv[k]], rows[k], s_g[k])

                @pl.when(c == 1)
                def _():
                    pltpu.async_copy(h1_hbm.at[srcv[k]], rows[k], s_g[k])

                @pl.when(i >= 1)
                def _():
                    pltpu.make_async_copy(
                        rows[o], acc.at[dstv[o]], s_s[o]).wait()

                @pl.when(i + 1 < ITERS)
                def _():
                    load_idx(i + 1, o)

                @pl.when(c == 0)
                def _():
                    pltpu.make_async_copy(
                        h0_hbm.at[srcv[k]], rows[k], s_g[k]).wait()

                @pl.when(c == 1)
                def _():
                    pltpu.make_async_copy(
                        h1_hbm.at[srcv[k]], rows[k], s_g[k]).wait()

                pltpu.async_copy(rows[k], acc.at[dstv[k]], s_s[k], add=True)

        pltpu.make_async_copy(rows[1], acc.at[dstv[1]], s_s[1]).wait()
        plsc.subcore_barrier()

        @pl.when(c == 0)
        def _():
            pltpu.sync_copy(acc.at[pl.ds(s * RPT, RPT)],
                            out0.at[pl.ds(s * RPT, RPT)])

        @pl.when(c == 1)
        def _():
            pltpu.sync_copy(acc.at[pl.ds(s * RPT, RPT)],
                            out1.at[pl.ds(s * RPT, RPT)])

    return agg


_agg = _make_agg()


def _init_body(x_ref, wlo_ref, whi_ref, ones_ref, blo_ref, bhi_ref,
               olo_ref, ohi_ref):
    x = x_ref[...]
    zl = jnp.dot(x, wlo_ref[...], preferred_element_type=jnp.float32)
    zh = jnp.dot(x, whi_ref[...], preferred_element_type=jnp.float32)
    zl = jnp.maximum(zl + blo_ref[...], 0.0)
    zh = jnp.maximum(zh + bhi_ref[...], 0.0)
    s = jnp.dot(zl * zl + zh * zh, ones_ref[...],
                preferred_element_type=jnp.float32)
    inv = 1.0 / jnp.maximum(jnp.sqrt(s), 1e-12)
    olo_ref[...] = zl * inv
    ohi_ref[...] = zh * inv


def _init_call(xp, wlo, whi, ones_bd, blo, bhi):
    return pl.pallas_call(
        _init_body,
        grid=(GRID,),
        in_specs=[
            pl.BlockSpec((BLK4, 8), lambda i: (i, 0)),
            pl.BlockSpec((8, 128), lambda i: (0, 0)),
            pl.BlockSpec((8, 128), lambda i: (0, 0)),
            pl.BlockSpec((128, 128), lambda i: (0, 0)),
            pl.BlockSpec((1, 128), lambda i: (0, 0)),
            pl.BlockSpec((1, 128), lambda i: (0, 0)),
        ],
        out_specs=[
            pl.BlockSpec((BLK4, 128), lambda i: (i, 0)),
            pl.BlockSpec((BLK4, 128), lambda i: (i, 0)),
        ],
        out_shape=[
            jax.ShapeDtypeStruct((RP4, 128), jnp.float32),
            jax.ShapeDtypeStruct((RP4, 128), jnp.float32),
        ],
    )(xp, wlo, whi, ones_bd, blo, bhi)


def _dense_body(hlo_ref, hhi_ref, alo_ref, ahi_ref, wlo_ref, whi_ref,
                ones_ref, blo_ref, bhi_ref, olo_ref, ohi_ref):
    x4 = jnp.concatenate(
        [hlo_ref[...], hhi_ref[...], alo_ref[...], ahi_ref[...]], axis=1)
    zl = jnp.dot(x4, wlo_ref[...], preferred_element_type=jnp.float32)
    zh = jnp.dot(x4, whi_ref[...], preferred_element_type=jnp.float32)
    zl = jnp.maximum(zl + blo_ref[...], 0.0)
    zh = jnp.maximum(zh + bhi_ref[...], 0.0)
    s = jnp.dot(zl * zl + zh * zh, ones_ref[...],
                preferred_element_type=jnp.float32)
    inv = 1.0 / jnp.maximum(jnp.sqrt(s), 1e-12)
    olo_ref[...] = zl * inv
    ohi_ref[...] = zh * inv


def _dense_call(h_lo, h_hi, a_lo, a_hi, wlo, whi, ones_bd, blo, bhi):
    return pl.pallas_call(
        _dense_body,
        grid=(GRID,),
        in_specs=[
            pl.BlockSpec((BLK4, 128), lambda i: (i, 0)),
            pl.BlockSpec((BLK4, 128), lambda i: (i, 0)),
            pl.BlockSpec((BLK4, 128), lambda i: (i, 0)),
            pl.BlockSpec((BLK4, 128), lambda i: (i, 0)),
            pl.BlockSpec((512, 128), lambda i: (0, 0)),
            pl.BlockSpec((512, 128), lambda i: (0, 0)),
            pl.BlockSpec((128, 128), lambda i: (0, 0)),
            pl.BlockSpec((1, 128), lambda i: (0, 0)),
            pl.BlockSpec((1, 128), lambda i: (0, 0)),
        ],
        out_specs=[
            pl.BlockSpec((BLK4, 128), lambda i: (i, 0)),
            pl.BlockSpec((BLK4, 128), lambda i: (i, 0)),
        ],
        out_shape=[
            jax.ShapeDtypeStruct((RP4, 128), jnp.float32),
            jax.ShapeDtypeStruct((RP4, 128), jnp.float32),
        ],
    )(h_lo, h_hi, a_lo, a_hi, wlo, whi, ones_bd, blo, bhi)


def kernel(input_features, input_feature_s, W1, W2, W3, lin_W, lin_b,
           edge_index, batch_ids, depth):
    f32 = jnp.float32
    dst = edge_index[0].astype(jnp.int32)
    src = edge_index[1].astype(jnp.int32)

    pad = EP - E_TOTAL
    seg_src = jnp.minimum(jnp.arange(N_NODES + pad, dtype=jnp.int32),
                          N_NODES - 1)
    seg_dst = jnp.concatenate([batch_ids.astype(jnp.int32) + N_NODES,
                               jnp.full((pad,), ROWS, jnp.int32)])
    src_full = jnp.concatenate(
        [src.reshape(E_EDGES // 128, 128),
         seg_src.reshape((N_NODES + pad) // 128, 128)]).reshape(EP)
    dst_full = jnp.concatenate(
        [dst.reshape(E_EDGES // 128, 128),
         seg_dst.reshape((N_NODES + pad) // 128, 128)]).reshape(EP)
    zeros = jnp.zeros((ROWS_PAD, H), f32)

    x_full = jnp.concatenate([
        input_features.astype(f32),
        input_feature_s.astype(f32),
        jnp.zeros((ROWS_PAD - ROWS, 2), f32),
    ])
    xp = x_full.reshape(RP4, 8)

    eye4 = jnp.eye(4, dtype=f32)

    def bd(k):
        return jnp.kron(eye4, k.astype(f32))

    W1f = W1.astype(f32)
    w1lo = bd(W1f[:, :H])
    w1hi = bd(W1f[:, H:])
    m2 = jnp.dot(W2.astype(f32), lin_W.astype(f32)[:D])
    m3 = jnp.dot(W3.astype(f32), lin_W.astype(f32)[D:])
    wlo = jnp.concatenate(
        [bd(m2[:H, :H]), bd(m2[H:, :H]), bd(m3[:H, :H]), bd(m3[H:, :H])],
        axis=0)
    whi = jnp.concatenate(
        [bd(m2[:H, H:]), bd(m2[H:, H:]), bd(m3[:H, H:]), bd(m3[H:, H:])],
        axis=0)
    ones_bd = bd(jnp.ones((H, H), f32))
    blo = jnp.tile(lin_b.astype(f32)[:H], 4).reshape(1, 128)
    bhi = jnp.tile(lin_b.astype(f32)[H:], 4).reshape(1, 128)
    zb = jnp.zeros((1, 128), f32)

    h_lo, h_hi = _init_call(xp, w1lo, w1hi, ones_bd, zb, zb)

    def step(_, carry):
        h_lo, h_hi = carry
        a_lo, a_hi = _agg(h_lo.reshape(ROWS_PAD, H), h_hi.reshape(ROWS_PAD, H),
                          src_full, dst_full, zeros)
        return tuple(_dense_call(h_lo, h_hi,
                                 a_lo.reshape(RP4, 128),
                                 a_hi.reshape(RP4, 128),
                                 wlo, whi, ones_bd, blo, bhi))

    h_lo, h_hi = lax.fori_loop(0, depth, step, (h_lo, h_hi))
    hh = jnp.concatenate([h_lo.reshape(ROWS_PAD, H),
                          h_hi.reshape(ROWS_PAD, H)], axis=1)
    return hh[:N_NODES], hh[N_NODES:ROWS]
